# baseline (speedup 1.0000x reference)
import jax
import jax.numpy as jnp
from jax.experimental import pallas as pl
from jax.experimental.pallas import tpu as pltpu

B, S, H, F, E = 32, 8, 1024, 4096, 16
N = B * S
RB = 32
NPAD = N + E * RB
FB = 2048
NF = F // FB


def _moe_kernel(x_ref, rwt_ref, w1_ref, w2_ref, out_ref,
                xs_ref, acc_ref, cnt_ref, off_ref, pos_ref):
    e = pl.program_id(0)
    f = pl.program_id(1)

    @pl.when((e == 0) & (f == 0))
    def _route_and_gather():
        x = x_ref[...]
        logits = jnp.dot(x, rwt_ref[...], preferred_element_type=jnp.float32)

        m = jnp.max(logits, axis=1, keepdims=True)
        ii = jax.lax.broadcasted_iota(jnp.int32, (N, E), 1)
        ids = jnp.min(jnp.where(logits == m, ii, E), axis=1,
                      keepdims=True)
        oh = (ii == ids).astype(jnp.float32)

        ones = jnp.full((N, 1), 1.0, jnp.float32)
        counts = jax.lax.dot_general(
            oh, ones, (((0,), (0,)), ((), ())),
            preferred_element_type=jnp.float32)
        counts_i = counts.astype(jnp.int32)
        padded = (((counts_i + RB - 1) // RB) * RB).astype(jnp.float32)
        a16 = jax.lax.broadcasted_iota(jnp.int32, (E, E), 0)
        b16 = jax.lax.broadcasted_iota(jnp.int32, (E, E), 1)
        mgt = (b16 < a16).astype(jnp.float32)
        offsets = jnp.dot(mgt, padded,
                          preferred_element_type=jnp.float32)

        ri = jax.lax.broadcasted_iota(jnp.int32, (N, N), 0)
        ci = jax.lax.broadcasted_iota(jnp.int32, (N, N), 1)
        lt = (ci < ri).astype(jnp.float32)
        csum = jnp.dot(lt, oh, preferred_element_type=jnp.float32)
        rank = jnp.sum(oh * csum, axis=1, keepdims=True)
        start = jnp.dot(oh, offsets,
                        preferred_element_type=jnp.float32)
        pos = (start + rank).astype(jnp.int32)

        cnt_ref[...] = counts_i
        off_ref[...] = offsets.astype(jnp.int32)
        pos_ref[...] = pos

        lane = jax.lax.broadcasted_iota(jnp.int32, (N, NPAD), 1)
        q = (lane == pos).astype(jnp.float32)
        xs_ref[...] = jax.lax.dot_general(
            q, x, (((0,), (0,)), ((), ())),
            preferred_element_type=jnp.float32)
        acc_ref[...] = jnp.zeros_like(acc_ref)
        out_ref[...] = jnp.zeros_like(out_ref)

    n = cnt_ref[e, 0]
    start = off_ref[e, 0]
    nb = (n + RB - 1) // RB
    w1 = w1_ref[0]
    w2 = w2_ref[0]

    def body(b, carry):
        row0 = pl.multiple_of(start + b * RB, 8)
        xb = xs_ref[pl.ds(row0, RB), :]
        h = jnp.dot(xb, w1, preferred_element_type=jnp.float32)
        h = h * jax.nn.sigmoid(h)
        c = jnp.dot(h, w2, preferred_element_type=jnp.float32)
        acc_ref[pl.ds(row0, RB), :] += c
        return carry

    jax.lax.fori_loop(0, nb, body, 0)

    @pl.when(f == NF - 1)
    def _unsort_expert():
        rel = pos_ref[...] - start
        lane = jax.lax.broadcasted_iota(jnp.int32, (N, N), 1)
        q2 = ((rel == lane) & (rel < nb * RB)).astype(jnp.float32)
        base = pl.multiple_of(start, 8)
        slab = acc_ref[pl.ds(base, N), :]
        out_ref[...] += jnp.dot(q2, slab,
                                preferred_element_type=jnp.float32)


@jax.jit
def kernel(hidden_states, router_w, W1, W2):
    x = hidden_states.reshape(N, H)
    rwt = router_w.T

    out = pl.pallas_call(
        _moe_kernel,
        grid=(E, NF),
        in_specs=[
            pl.BlockSpec((N, H), lambda e, f: (0, 0)),
            pl.BlockSpec((H, E), lambda e, f: (0, 0)),
            pl.BlockSpec((1, H, FB), lambda e, f: (e, 0, f)),
            pl.BlockSpec((1, FB, H), lambda e, f: (e, f, 0)),
        ],
        out_specs=pl.BlockSpec((N, H), lambda e, f: (0, 0)),
        scratch_shapes=[
            pltpu.VMEM((NPAD, H), jnp.float32),
            pltpu.VMEM((NPAD + N, H), jnp.float32),
            pltpu.VMEM((E, 1), jnp.int32),
            pltpu.VMEM((E, 1), jnp.int32),
            pltpu.VMEM((N, 1), jnp.int32),
        ],
        out_shape=jax.ShapeDtypeStruct((N, H), jnp.float32),
    )(x, rwt, W1, W2)

    return out.reshape(B, S, H)


# speedup vs baseline: 1.0024x; 1.0024x over previous
import jax
import jax.numpy as jnp
from jax.experimental import pallas as pl
from jax.experimental.pallas import tpu as pltpu

B, S, H, F, E = 32, 8, 1024, 4096, 16
N = B * S
RB = 32
NPAD = N + E * RB
FB = 2048
NF = F // FB


def _moe_kernel(x_ref, rwt_ref, w1_ref, w2_ref, out_ref,
                xs_ref, acc_ref, cnt_ref, off_ref, pos_ref):
    e = pl.program_id(0)
    f = pl.program_id(1)

    @pl.when((e == 0) & (f == 0))
    def _route_and_gather():
        x = x_ref[...]
        logits = jnp.dot(x, rwt_ref[...], preferred_element_type=jnp.float32)

        m = jnp.max(logits, axis=1, keepdims=True)
        ii = jax.lax.broadcasted_iota(jnp.int32, (N, E), 1)
        ids = jnp.min(jnp.where(logits == m, ii, E), axis=1,
                      keepdims=True)
        oh = (ii == ids).astype(jnp.float32)

        ones = jnp.full((N, 1), 1.0, jnp.float32)
        counts = jax.lax.dot_general(
            oh, ones, (((0,), (0,)), ((), ())),
            preferred_element_type=jnp.float32)
        counts_i = counts.astype(jnp.int32)
        padded = (((counts_i + RB - 1) // RB) * RB).astype(jnp.float32)
        a16 = jax.lax.broadcasted_iota(jnp.int32, (E, E), 0)
        b16 = jax.lax.broadcasted_iota(jnp.int32, (E, E), 1)
        mgt = (b16 < a16).astype(jnp.float32)
        offsets = jnp.dot(mgt, padded,
                          preferred_element_type=jnp.float32)

        ri = jax.lax.broadcasted_iota(jnp.int32, (N, N), 0)
        ci = jax.lax.broadcasted_iota(jnp.int32, (N, N), 1)
        lt = (ci < ri).astype(jnp.float32)
        csum = jnp.dot(lt, oh, preferred_element_type=jnp.float32)
        rank = jnp.sum(oh * csum, axis=1, keepdims=True)
        start = jnp.dot(oh, offsets,
                        preferred_element_type=jnp.float32)
        pos = (start + rank).astype(jnp.int32)

        cnt_ref[...] = counts_i
        off_ref[...] = offsets.astype(jnp.int32)
        pos_ref[...] = pos

        lane = jax.lax.broadcasted_iota(jnp.int32, (N, NPAD), 1)
        q = (lane == pos).astype(jnp.float32)
        xs_ref[...] = jax.lax.dot_general(
            q, x, (((0,), (0,)), ((), ())),
            preferred_element_type=jnp.float32)
        acc_ref[...] = jnp.zeros_like(acc_ref)

    n = cnt_ref[e, 0]
    start = off_ref[e, 0]
    nb = (n + RB - 1) // RB
    w1 = w1_ref[0]
    w2 = w2_ref[0]

    def body(b, carry):
        row0 = pl.multiple_of(start + b * RB, 8)
        xb = xs_ref[pl.ds(row0, RB), :]
        h = jnp.dot(xb, w1, preferred_element_type=jnp.float32)
        h = h * jax.nn.sigmoid(h)
        c = jnp.dot(h, w2, preferred_element_type=jnp.float32)
        acc_ref[pl.ds(row0, RB), :] += c
        return carry

    jax.lax.fori_loop(0, nb, body, 0)

    @pl.when((e == E - 1) & (f == NF - 1))
    def _unsort():
        lane = jax.lax.broadcasted_iota(jnp.int32, (N, NPAD), 1)
        q2 = (lane == pos_ref[...]).astype(jnp.float32)
        out_ref[...] = jnp.dot(q2, acc_ref[...],
                               preferred_element_type=jnp.float32)


@jax.jit
def kernel(hidden_states, router_w, W1, W2):
    x = hidden_states.reshape(N, H)
    rwt = router_w.T

    out = pl.pallas_call(
        _moe_kernel,
        grid=(E, NF),
        in_specs=[
            pl.BlockSpec((N, H), lambda e, f: (0, 0)),
            pl.BlockSpec((H, E), lambda e, f: (0, 0)),
            pl.BlockSpec((1, H, FB), lambda e, f: (e, 0, f)),
            pl.BlockSpec((1, FB, H), lambda e, f: (e, f, 0)),
        ],
        out_specs=pl.BlockSpec((N, H), lambda e, f: (0, 0)),
        scratch_shapes=[
            pltpu.VMEM((NPAD, H), jnp.float32),
            pltpu.VMEM((NPAD, H), jnp.float32),
            pltpu.VMEM((E, 1), jnp.int32),
            pltpu.VMEM((E, 1), jnp.int32),
            pltpu.VMEM((N, 1), jnp.int32),
        ],
        out_shape=jax.ShapeDtypeStruct((N, H), jnp.float32),
    )(x, rwt, W1, W2)

    return out.reshape(B, S, H)
